# baseline (speedup 1.0000x reference)
import functools

import jax
import jax.numpy as jnp
from jax import lax
from jax.experimental import pallas as pl
from jax.experimental.pallas import tpu as pltpu
from jax.experimental.pallas import tpu_sc as plsc

BATCH = 4096
SEQ_LEN = 200
EMB = 64

NUM_CORES = 2
NUM_SUBCORES = 16
NUM_WORKERS = NUM_CORES * NUM_SUBCORES

TOTAL = BATCH * SEQ_LEN
PER_WORKER = TOTAL // NUM_WORKERS
CHUNK = 128
CHUNKS_PER_WORKER = PER_WORKER // CHUNK


def _make_kernel(vocab):
    mesh = plsc.VectorSubcoreMesh(core_axis_name="c", subcore_axis_name="s")

    @functools.partial(
        pl.kernel,
        out_type=jax.ShapeDtypeStruct((TOTAL, EMB), jnp.float32),
        mesh=mesh,
        scratch_types=[
            pltpu.VMEM((CHUNK,), jnp.int32),
            pltpu.VMEM((CHUNK, EMB), jnp.float32),
            pltpu.VMEM((2 * SEQ_LEN, EMB), jnp.float32),
            pltpu.SemaphoreType.DMA,
        ],
        compiler_params=pltpu.CompilerParams(use_tc_tiling_on_sc=False),
    )
    def emb_kernel(idx_hbm, wt_hbm, pos_hbm, out_hbm, idx_v, rows_v, pos_v, sem):
        wid = lax.axis_index("s") * NUM_CORES + lax.axis_index("c")
        base = wid * PER_WORKER

        pltpu.sync_copy(pos_hbm, pos_v.at[pl.ds(0, SEQ_LEN)])
        pltpu.sync_copy(pos_hbm, pos_v.at[pl.ds(SEQ_LEN, SEQ_LEN)])

        @pl.loop(0, CHUNKS_PER_WORKER)
        def _chunk(k):
            start = base + k * CHUNK
            phase = lax.rem(k * CHUNK, SEQ_LEN)
            pltpu.sync_copy(idx_hbm.at[pl.ds(start, CHUNK)], idx_v)
            pltpu.async_copy(wt_hbm.at[idx_v], rows_v, sem).wait()

            @pl.loop(0, CHUNK)
            def _row(j):
                p = phase + j
                for s in range(EMB // 16):
                    sl = pl.ds(s * 16, 16)
                    rows_v[j, sl] = rows_v[j, sl] + pos_v[p, sl]

            pltpu.sync_copy(rows_v, out_hbm.at[pl.ds(start, CHUNK)])

    return emb_kernel


def kernel(inputs, word_table, pos_table):
    vocab = word_table.shape[0]
    idx_flat = inputs.reshape(TOTAL).astype(jnp.int32)
    out_flat = _make_kernel(vocab)(idx_flat, word_table, pos_table)
    return out_flat.reshape(BATCH, SEQ_LEN, EMB)


# speedup vs baseline: 1.2148x; 1.2148x over previous
import functools

import jax
import jax.numpy as jnp
from jax import lax
from jax.experimental import pallas as pl
from jax.experimental.pallas import tpu as pltpu
from jax.experimental.pallas import tpu_sc as plsc

BATCH = 4096
SEQ_LEN = 200
EMB = 64

NUM_CORES = 2
NUM_SUBCORES = 16
NUM_WORKERS = NUM_CORES * NUM_SUBCORES

TOTAL = BATCH * SEQ_LEN
PER_WORKER = TOTAL // NUM_WORKERS
CHUNK = 128
CHUNKS_PER_WORKER = PER_WORKER // CHUNK
NBUF = 4
PREFETCH = 2


def _make_kernel():
    mesh = plsc.VectorSubcoreMesh(core_axis_name="c", subcore_axis_name="s")

    @functools.partial(
        pl.kernel,
        out_type=jax.ShapeDtypeStruct((TOTAL, EMB), jnp.float32),
        mesh=mesh,
        scratch_types=[
            pltpu.VMEM((CHUNKS_PER_WORKER, CHUNK), jnp.int32),
            pltpu.VMEM((NBUF, CHUNK, EMB), jnp.float32),
            pltpu.VMEM((2 * SEQ_LEN, EMB), jnp.float32),
            [pltpu.SemaphoreType.DMA] * NBUF,
            [pltpu.SemaphoreType.DMA] * NBUF,
        ],
        compiler_params=pltpu.CompilerParams(use_tc_tiling_on_sc=False),
    )
    def emb_kernel(idx_hbm, wt_hbm, pos_hbm, out_hbm,
                   idx_v, rows_v, pos_v, g_sems, o_sems):
        wid = lax.axis_index("s") * NUM_CORES + lax.axis_index("c")
        base = wid * PER_WORKER

        pltpu.sync_copy(idx_hbm.at[pl.ds(wid * CHUNKS_PER_WORKER,
                                         CHUNKS_PER_WORKER)], idx_v)
        pltpu.sync_copy(pos_hbm, pos_v.at[pl.ds(0, SEQ_LEN)])
        pltpu.sync_copy(pos_hbm, pos_v.at[pl.ds(SEQ_LEN, SEQ_LEN)])

        def start_gather(g, b):
            pltpu.async_copy(wt_hbm.at[idx_v.at[g]], rows_v.at[b], g_sems[b])

        for b in range(PREFETCH):
            start_gather(b, b)

        @pl.loop(0, CHUNKS_PER_WORKER, step=NBUF)
        def _block(k):
            for b in range(NBUF):
                g = k + b
                pltpu.make_async_copy(wt_hbm.at[idx_v.at[g]],
                                      rows_v.at[b], g_sems[b]).wait()
                phase = lax.rem(g * CHUNK, SEQ_LEN)

                @pl.loop(0, CHUNK)
                def _row(j):
                    p = phase + j
                    for s in range(EMB // 16):
                        sl = pl.ds(s * 16, 16)
                        rows_v[b, j, sl] = rows_v[b, j, sl] + pos_v[p, sl]

                pltpu.async_copy(rows_v.at[b],
                                 out_hbm.at[pl.ds(base + g * CHUNK, CHUNK)],
                                 o_sems[b])

                t = g + PREFETCH
                tb = (b + PREFETCH) % NBUF

                @pl.when(t < CHUNKS_PER_WORKER)
                def _():
                    @pl.when(t >= NBUF)
                    def _():
                        pltpu.make_async_copy(
                            rows_v.at[tb],
                            out_hbm.at[pl.ds(base + (t - NBUF) * CHUNK, CHUNK)],
                            o_sems[tb]).wait()
                    start_gather(t, tb)

        for g in range(CHUNKS_PER_WORKER - NBUF, CHUNKS_PER_WORKER):
            b = g % NBUF
            pltpu.make_async_copy(rows_v.at[b],
                                  out_hbm.at[pl.ds(base + g * CHUNK, CHUNK)],
                                  o_sems[b]).wait()

    return emb_kernel


def kernel(inputs, word_table, pos_table):
    idx_blocks = inputs.reshape(TOTAL // CHUNK, CHUNK).astype(jnp.int32)
    out_flat = _make_kernel()(idx_blocks, word_table, pos_table)
    return out_flat.reshape(BATCH, SEQ_LEN, EMB)
